# Initial kernel scaffold; baseline (speedup 1.0000x reference)
import jax
import jax.numpy as jnp
from jax.experimental import pallas as pl


def kernel(l_size, c_size, l_edge_index, c_edge_index, l_emb, c_emb, l2c_W1, l2c_b1, l2c_W2, l2c_b2, c2l_W1, c2l_b1, c2l_W2, c2l_b2, c_upd_W, c_upd_b, l_upd_W, l_upd_b):
    raise NotImplementedError("write your pallas kernel here")


# speedup vs baseline: 6.4854x; 6.4854x over previous
import functools

import jax
import jax.numpy as jnp
from jax import lax
from jax.experimental import pallas as pl
from jax.experimental.pallas import tpu as pltpu
from jax.experimental.pallas import tpu_sc as plsc

NC = 2
NS = 16
GROUP = 128
SB = 8
ZR = 128
WAVE = 4
CHUNK = 32


def _pad_to(x, n, value):
    return jnp.concatenate(
        [x, jnp.full((n - x.shape[0],), value, dtype=x.dtype)])


def _sc_mesh():
    return plsc.VectorSubcoreMesh(
        core_axis_name="c", subcore_axis_name="s", num_cores=NC,
        num_subcores=NS)


def _staged_copy(src_at, dst_at, rows):
    off = 0
    while off < rows:
        sz = min(ZR, rows - off)
        pltpu.sync_copy(src_at(off, sz), dst_at(off, sz))
        off += sz


def _agg_pass(sid, msg_ref, src2d, dst2d, out_ref, n_pad, n_groups,
              acc, srcbuf, dstbuf, rowbuf, zbuf, sem):
    rpt = n_pad // NS
    base = pl.multiple_of(sid * rpt, 8)
    _staged_copy(lambda o, s: zbuf.at[pl.ds(0, s)],
                 lambda o, s: acc.at[pl.ds(base + o, s)], rpt)
    plsc.subcore_barrier()

    gpt = n_groups // NS
    g0 = sid * gpt

    def block(b, carry):
        gbase = pl.multiple_of(g0 + b * SB, 8)
        pltpu.sync_copy(src2d.at[pl.ds(gbase, SB)], srcbuf)
        pltpu.sync_copy(dst2d.at[pl.ds(gbase, SB)], dstbuf)
        for w in range(SB // WAVE):
            descs = [
                pltpu.async_copy(msg_ref.at[srcbuf.at[w * WAVE + j]],
                                 rowbuf.at[j], sem)
                for j in range(WAVE)
            ]
            for d in descs:
                d.wait()
            for j in range(WAVE):
                pltpu.sync_copy(rowbuf.at[j], acc.at[dstbuf.at[w * WAVE + j]],
                                add=True)
        return carry

    lax.fori_loop(0, gpt // SB, block, 0)
    plsc.subcore_barrier()
    _staged_copy(lambda o, s: acc.at[pl.ds(base + o, s)],
                 lambda o, s: out_ref.at[pl.ds(base + o, s)], rpt)
    plsc.subcore_barrier()


def _make_agg_call(n_l, n_c, nl_pad, nc_pad, n_groups):
    def body(ml0, ml1, ml2, ml3, mc0, mc1, mc2, mc3,
             lsrc, ldst, csrc, cdst, zeros_hbm,
             ac0, ac1, ac2, ac3, al0, al1, al2, al3,
             acc, srcbuf, dstbuf, rowbuf, zbuf, sem):
        cc = lax.axis_index("c")
        sid = lax.axis_index("s")
        pltpu.sync_copy(zeros_hbm, zbuf)

        def run(msg_ref, src2d, dst2d, out_ref, n_pad):
            _agg_pass(sid, msg_ref, src2d, dst2d, out_ref, n_pad, n_groups,
                      acc, srcbuf, dstbuf, rowbuf, zbuf, sem)

        @pl.when(cc == 0)
        def _():
            run(ml0, lsrc, cdst, ac0, nc_pad)
            run(ml1, lsrc, cdst, ac1, nc_pad)
            run(mc0, csrc, ldst, al0, nl_pad)
            run(mc1, csrc, ldst, al1, nl_pad)

        @pl.when(cc == 1)
        def _():
            run(ml2, lsrc, cdst, ac2, nc_pad)
            run(ml3, lsrc, cdst, ac3, nc_pad)
            run(mc2, csrc, ldst, al2, nl_pad)
            run(mc3, csrc, ldst, al3, nl_pad)

    out_c = jax.ShapeDtypeStruct((nc_pad, CHUNK), jnp.float32)
    out_l = jax.ShapeDtypeStruct((nl_pad, CHUNK), jnp.float32)
    return pl.kernel(
        body,
        out_type=(out_c,) * 4 + (out_l,) * 4,
        mesh=_sc_mesh(),
        compiler_params=pltpu.CompilerParams(use_tc_tiling_on_sc=False),
        scratch_types=[
            pltpu.VMEM_SHARED((nc_pad, CHUNK), jnp.float32),
            pltpu.VMEM((SB, GROUP), jnp.int32),
            pltpu.VMEM((SB, GROUP), jnp.int32),
            pltpu.VMEM((WAVE, GROUP, CHUNK), jnp.float32),
            pltpu.VMEM((ZR, CHUNK), jnp.float32),
            pltpu.SemaphoreType.DMA,
        ],
    )


def _deg_pass(sid, dst2d, out_ref, n_pad, n_groups, acc, dstbuf, onesbuf,
              zbuf):
    rpt = n_pad // NS
    base = pl.multiple_of(sid * rpt, 8)
    _staged_copy(lambda o, s: zbuf.at[pl.ds(0, s)],
                 lambda o, s: acc.at[pl.ds(base + o, s)], rpt)
    plsc.subcore_barrier()
    gpt = n_groups // NS
    g0 = sid * gpt

    def block(b, carry):
        gbase = pl.multiple_of(g0 + b * SB, 8)
        pltpu.sync_copy(dst2d.at[pl.ds(gbase, SB)], dstbuf)
        for j in range(SB):
            pltpu.sync_copy(onesbuf, acc.at[dstbuf.at[j]], add=True)
        return carry

    lax.fori_loop(0, gpt // SB, block, 0)
    plsc.subcore_barrier()
    _staged_copy(lambda o, s: acc.at[pl.ds(base + o, s)],
                 lambda o, s: out_ref.at[pl.ds(base + o, s)], rpt)


def _make_deg_call(nl_pad, nc_pad, n_groups):
    def body(ldst, cdst, ones_hbm, zeros_hbm, dc, dl,
             acc, dstbuf, onesbuf, zbuf):
        cc = lax.axis_index("c")
        sid = lax.axis_index("s")
        pltpu.sync_copy(ones_hbm, onesbuf)
        pltpu.sync_copy(zeros_hbm, zbuf)

        @pl.when(cc == 0)
        def _():
            _deg_pass(sid, cdst, dc, nc_pad, n_groups, acc, dstbuf, onesbuf,
                      zbuf)

        @pl.when(cc == 1)
        def _():
            _deg_pass(sid, ldst, dl, nl_pad, n_groups, acc, dstbuf, onesbuf,
                      zbuf)

    return pl.kernel(
        body,
        out_type=(jax.ShapeDtypeStruct((nc_pad, 16), jnp.float32),
                  jax.ShapeDtypeStruct((nl_pad, 16), jnp.float32)),
        mesh=_sc_mesh(),
        compiler_params=pltpu.CompilerParams(use_tc_tiling_on_sc=False),
        scratch_types=[
            pltpu.VMEM_SHARED((nc_pad, 16), jnp.float32),
            pltpu.VMEM((SB, GROUP), jnp.int32),
            pltpu.VMEM((GROUP, 16), jnp.float32),
            pltpu.VMEM((ZR, 16), jnp.float32),
        ],
    )


def _msg_body(x_ref, w1_ref, b1_ref, w2_ref, b2_ref, deg_ref,
              o0, o1, o2, o3):
    x = x_ref[...]
    h = jnp.maximum(
        jnp.dot(x, w1_ref[...], preferred_element_type=jnp.float32)
        + b1_ref[...], 0.0)
    y = (jnp.dot(h, w2_ref[...], preferred_element_type=jnp.float32)
         + b2_ref[...])
    y = y * lax.rsqrt(jnp.maximum(deg_ref[...], 1.0))
    o0[...] = y[:, 0 * CHUNK:1 * CHUNK]
    o1[...] = y[:, 1 * CHUNK:2 * CHUNK]
    o2[...] = y[:, 2 * CHUNK:3 * CHUNK]
    o3[...] = y[:, 3 * CHUNK:4 * CHUNK]


def _msg_call(x, w1, b1, w2, b2, deg, block_rows):
    n, d = x.shape
    grid = (n // block_rows,)
    full = lambda i: (0, 0)
    rowb = lambda i: (i, 0)
    return pl.pallas_call(
        _msg_body,
        grid=grid,
        in_specs=[
            pl.BlockSpec((block_rows, d), rowb),
            pl.BlockSpec((d, d), full),
            pl.BlockSpec((1, d), full),
            pl.BlockSpec((d, d), full),
            pl.BlockSpec((1, d), full),
            pl.BlockSpec((block_rows, 1), rowb),
        ],
        out_specs=[pl.BlockSpec((block_rows, CHUNK), rowb)] * 4,
        out_shape=[jax.ShapeDtypeStruct((n, CHUNK), jnp.float32)] * 4,
    )(x, w1, b1.reshape(1, d), w2, b2.reshape(1, d), deg)


def _c_upd_body(x_ref, agg_ref, deg_ref, wa_ref, wb_ref, b_ref, o_ref):
    s = lax.rsqrt(jnp.maximum(deg_ref[...], 1.0))
    o_ref[...] = (
        jnp.dot(x_ref[...], wa_ref[...], preferred_element_type=jnp.float32)
        + jnp.dot(agg_ref[...] * s, wb_ref[...],
                  preferred_element_type=jnp.float32)
        + b_ref[...])


def _c_upd_call(x, agg, deg, wa, wb, b, block_rows):
    n, d = x.shape
    grid = (n // block_rows,)
    full = lambda i: (0, 0)
    rowb = lambda i: (i, 0)
    return pl.pallas_call(
        _c_upd_body,
        grid=grid,
        in_specs=[
            pl.BlockSpec((block_rows, d), rowb),
            pl.BlockSpec((block_rows, d), rowb),
            pl.BlockSpec((block_rows, 1), rowb),
            pl.BlockSpec((d, d), full),
            pl.BlockSpec((d, d), full),
            pl.BlockSpec((1, d), full),
        ],
        out_specs=pl.BlockSpec((block_rows, d), rowb),
        out_shape=jax.ShapeDtypeStruct((n, d), jnp.float32),
    )(x, agg, deg, wa, wb, b.reshape(1, d))


def _l_upd_body(x_ref, agg_ref, sw_ref, deg_ref, wa_ref, wb_ref, wc_ref,
                b_ref, o_ref):
    s = lax.rsqrt(jnp.maximum(deg_ref[...], 1.0))
    o_ref[...] = (
        jnp.dot(x_ref[...], wa_ref[...], preferred_element_type=jnp.float32)
        + jnp.dot(agg_ref[...] * s, wb_ref[...],
                  preferred_element_type=jnp.float32)
        + jnp.dot(sw_ref[...], wc_ref[...],
                  preferred_element_type=jnp.float32)
        + b_ref[...])


def _l_upd_call(x, agg, sw, deg, wa, wb, wc, b, block_rows):
    n, d = x.shape
    grid = (n // block_rows,)
    full = lambda i: (0, 0)
    rowb = lambda i: (i, 0)
    return pl.pallas_call(
        _l_upd_body,
        grid=grid,
        in_specs=[
            pl.BlockSpec((block_rows, d), rowb),
            pl.BlockSpec((block_rows, d), rowb),
            pl.BlockSpec((block_rows, d), rowb),
            pl.BlockSpec((block_rows, 1), rowb),
            pl.BlockSpec((d, d), full),
            pl.BlockSpec((d, d), full),
            pl.BlockSpec((d, d), full),
            pl.BlockSpec((1, d), full),
        ],
        out_specs=pl.BlockSpec((block_rows, d), rowb),
        out_shape=jax.ShapeDtypeStruct((n, d), jnp.float32),
    )(x, agg, sw, deg, wa, wb, wc, b.reshape(1, d))


def kernel(l_size, c_size, l_edge_index, c_edge_index, l_emb, c_emb,
           l2c_W1, l2c_b1, l2c_W2, l2c_b2,
           c2l_W1, c2l_b1, c2l_W2, c2l_b2,
           c_upd_W, c_upd_b, l_upd_W, l_upd_b):
    n_l, d = l_emb.shape
    n_c = c_emb.shape[0]
    e = l_edge_index.shape[0]
    n_iter = 4

    unit = NS * GROUP * SB
    e_pad = ((e + unit - 1) // unit) * unit
    n_groups = e_pad // GROUP
    li = l_edge_index.astype(jnp.int32)
    ci = c_edge_index.astype(jnp.int32)
    lsrc = _pad_to(li, e_pad, 0).reshape(n_groups, GROUP)
    ldst = _pad_to(li, e_pad, n_l).reshape(n_groups, GROUP)
    csrc = _pad_to(ci, e_pad, 0).reshape(n_groups, GROUP)
    cdst = _pad_to(ci, e_pad, n_c).reshape(n_groups, GROUP)

    nl_pad = ((n_l + 1 + NS * 8 - 1) // (NS * 8)) * (NS * 8)
    nc_pad = ((n_c + 1 + NS * 8 - 1) // (NS * 8)) * (NS * 8)

    zeros32 = jnp.zeros((ZR, CHUNK), jnp.float32)
    zeros16 = jnp.zeros((ZR, 16), jnp.float32)
    ones16 = jnp.ones((GROUP, 16), jnp.float32)

    deg_call = _make_deg_call(nl_pad, nc_pad, n_groups)
    dc_pad, dl_pad = deg_call(ldst, cdst, ones16, zeros16)
    l_deg = dl_pad[:n_l, 0:1]
    c_deg = dc_pad[:n_c, 0:1]

    agg_call = _make_agg_call(n_l, n_c, nl_pad, nc_pad, n_groups)

    c_wa, c_wb = c_upd_W[:d], c_upd_W[d:]
    l_wa, l_wb, l_wc = l_upd_W[:d], l_upd_W[d:2 * d], l_upd_W[2 * d:]

    l_blk, c_blk = 800, 2000
    l_embs = [l_emb]
    c_embs = [c_emb]
    for _ in range(n_iter):
        ml = _msg_call(l_emb, l2c_W1, l2c_b1, l2c_W2, l2c_b2, l_deg, l_blk)
        mc = _msg_call(c_emb, c2l_W1, c2l_b1, c2l_W2, c2l_b2, c_deg, c_blk)
        outs = agg_call(*ml, *mc, lsrc, ldst, csrc, cdst, zeros32)
        agg_c = jnp.concatenate(outs[0:4], axis=1)[:n_c]
        agg_l = jnp.concatenate(outs[4:8], axis=1)[:n_l]

        lswap = l_emb.reshape(n_l // 2, 2, d)[:, ::-1, :].reshape(n_l, d)
        c_emb = _c_upd_call(c_emb, agg_c, c_deg, c_wa, c_wb, c_upd_b, c_blk)
        l_emb = _l_upd_call(l_emb, agg_l, lswap, l_deg, l_wa, l_wb, l_wc,
                            l_upd_b, l_blk)
        c_embs.append(c_emb)
        l_embs.append(l_emb)

    return (jnp.stack(l_embs), jnp.stack(c_embs))
